# initial kernel scaffold (unmeasured)
import jax
import jax.numpy as jnp
from jax import lax
from jax.experimental import pallas as pl
from jax.experimental.pallas import tpu as pltpu

N_DEV = 4


def kernel(x, w_mat, scale_x, scale_w):
    m, k_shard = x.shape
    _, n = w_mat.shape
    m_chunk = m // N_DEV

    def body(x_ref, w_ref, sx_ref, sw_ref, out_ref, comm_ref,
             send_sems, recv_sems):
        my = lax.axis_index("i")
        left = (my + N_DEV - 1) % N_DEV
        right = (my + 1) % N_DEV

        barrier_sem = pltpu.get_barrier_semaphore()
        for nbr in (left, right):
            pl.semaphore_signal(
                barrier_sem, inc=1,
                device_id=(nbr,), device_id_type=pl.DeviceIdType.MESH,
            )
        pl.semaphore_wait(barrier_sem, 2)

        xb = x_ref[...].astype(jnp.bfloat16)
        wb = w_ref[...].astype(jnp.bfloat16)
        out_ref[...] = lax.dot_general(
            xb, wb, (((1,), (0,)), ((), ())),
            preferred_element_type=jnp.float32,
        )


        for s in range(N_DEV - 1):
            send_c = (my - s) % N_DEV
            recv_c = (my - s - 1) % N_DEV
            slot = s % 4
            rdma = pltpu.make_async_remote_copy(
                src_ref=out_ref.at[pl.ds(send_c * m_chunk, m_chunk)],
                dst_ref=comm_ref.at[slot],
                send_sem=send_sems.at[s],
                recv_sem=recv_sems.at[s],
                device_id=(right,),
                device_id_type=pl.DeviceIdType.MESH,
            )
            rdma.start()
            rdma.wait()
            out_ref[pl.ds(recv_c * m_chunk, m_chunk), :] = (
                out_ref[pl.ds(recv_c * m_chunk, m_chunk), :]
                + comm_ref[slot]
            )

        for t in range(N_DEV - 1):
            h = N_DEV - 1 + t
            send_c = (my + 1 - t) % N_DEV
            recv_c = (my - t) % N_DEV
            slot = h % 4
            rdma = pltpu.make_async_remote_copy(
                src_ref=out_ref.at[pl.ds(send_c * m_chunk, m_chunk)],
                dst_ref=comm_ref.at[slot],
                send_sem=send_sems.at[h],
                recv_sem=recv_sems.at[h],
                device_id=(right,),
                device_id_type=pl.DeviceIdType.MESH,
            )
            rdma.start()
            rdma.wait()
            out_ref[pl.ds(recv_c * m_chunk, m_chunk), :] = comm_ref[slot]

        scale = sx_ref[0] * sw_ref[0]
        y = out_ref[...] * scale
        out_ref[...] = y / (1.0 + jnp.exp(-jnp.clip(y, -60.0, 60.0)))

    return pl.pallas_call(
        body,
        out_shape=jax.ShapeDtypeStruct((m, n), jnp.float32),
        in_specs=[
            pl.BlockSpec(memory_space=pltpu.VMEM),
            pl.BlockSpec(memory_space=pltpu.VMEM),
            pl.BlockSpec(memory_space=pltpu.SMEM),
            pl.BlockSpec(memory_space=pltpu.SMEM),
        ],
        out_specs=pl.BlockSpec(memory_space=pltpu.VMEM),
        scratch_shapes=[
            pltpu.VMEM((4, m_chunk, n), jnp.float32),
            pltpu.SemaphoreType.DMA((2 * (N_DEV - 1),)),
            pltpu.SemaphoreType.DMA((2 * (N_DEV - 1),)),
        ],
        compiler_params=pltpu.CompilerParams(collective_id=0),
    )(x, w_mat, scale_x, scale_w)


# baseline (device time: 592085 ns/iter reference)
import jax
import jax.numpy as jnp
from jax import lax
from jax.experimental import pallas as pl
from jax.experimental.pallas import tpu as pltpu

N_DEV = 4


def kernel(x, w_mat, scale_x, scale_w):
    m, k_shard = x.shape
    _, n = w_mat.shape
    m_chunk = m // N_DEV

    def body(x_ref, w_ref, sx_ref, sw_ref, out_ref,
             wb_ref, acc_ref, comm_ref, send_sems, recv_sems, copy_sem):
        my = lax.axis_index("i")
        left = (my + N_DEV - 1) % N_DEV
        right = (my + 1) % N_DEV

        barrier_sem = pltpu.get_barrier_semaphore()
        for nbr in (left, right):
            pl.semaphore_signal(
                barrier_sem, inc=1,
                device_id=(nbr,), device_id_type=pl.DeviceIdType.MESH,
            )
        pl.semaphore_wait(barrier_sem, 2)

        wb_ref[...] = w_ref[...].astype(jnp.bfloat16)

        def partial_gemm(c, buf):
            xb = x_ref[pl.ds(c * m_chunk, m_chunk), :].astype(jnp.bfloat16)
            acc_ref[buf] = lax.dot_general(
                xb, wb_ref[...], (((1,), (0,)), ((), ())),
                preferred_element_type=jnp.float32,
            )

        partial_gemm(my, 0)
        for s in range(N_DEV - 1):
            send_buf = s % 2
            work_buf = (s + 1) % 2
            rdma = pltpu.make_async_remote_copy(
                src_ref=acc_ref.at[send_buf],
                dst_ref=comm_ref.at[s],
                send_sem=send_sems.at[s],
                recv_sem=recv_sems.at[s],
                device_id=(right,),
                device_id_type=pl.DeviceIdType.MESH,
            )
            rdma.start()
            partial_gemm((my - s - 1) % N_DEV, work_buf)
            rdma.wait()
            acc_ref[work_buf] = acc_ref[work_buf] + comm_ref[s]

        final_buf = (N_DEV - 1) % 2
        own_c = (my + 1) % N_DEV

        scale = sx_ref[0] * sw_ref[0]
        y = acc_ref[final_buf] * scale
        acc_ref[final_buf] = y / (1.0 + jnp.exp(-jnp.clip(y, -60.0, 60.0)))

        local_cp = pltpu.make_async_copy(
            acc_ref.at[final_buf],
            out_ref.at[pl.ds(own_c * m_chunk, m_chunk)],
            copy_sem,
        )
        local_cp.start()
        local_cp.wait()

        for t in range(N_DEV - 1):
            h = N_DEV - 1 + t
            send_c = (my + 1 - t) % N_DEV
            rdma = pltpu.make_async_remote_copy(
                src_ref=out_ref.at[pl.ds(send_c * m_chunk, m_chunk)],
                dst_ref=out_ref.at[pl.ds(send_c * m_chunk, m_chunk)],
                send_sem=send_sems.at[h],
                recv_sem=recv_sems.at[h],
                device_id=(right,),
                device_id_type=pl.DeviceIdType.MESH,
            )
            rdma.start()
            rdma.wait()

    return pl.pallas_call(
        body,
        out_shape=jax.ShapeDtypeStruct((m, n), jnp.float32),
        in_specs=[
            pl.BlockSpec(memory_space=pltpu.VMEM),
            pl.BlockSpec(memory_space=pltpu.VMEM),
            pl.BlockSpec(memory_space=pltpu.SMEM),
            pl.BlockSpec(memory_space=pltpu.SMEM),
        ],
        out_specs=pl.BlockSpec(memory_space=pltpu.MemorySpace.HBM),
        scratch_shapes=[
            pltpu.VMEM((k_shard, n), jnp.bfloat16),
            pltpu.VMEM((2, m_chunk, n), jnp.float32),
            pltpu.VMEM((N_DEV - 1, m_chunk, n), jnp.float32),
            pltpu.SemaphoreType.DMA((2 * (N_DEV - 1),)),
            pltpu.SemaphoreType.DMA((2 * (N_DEV - 1),)),
            pltpu.SemaphoreType.DMA,
        ],
        compiler_params=pltpu.CompilerParams(
            collective_id=0,
            vmem_limit_bytes=56 * 1024 * 1024,
        ),
    )(x, w_mat, scale_x, scale_w)


# device time: 319278 ns/iter; 1.8544x vs baseline; 1.8544x over previous
import jax
import jax.numpy as jnp
from jax import lax
from jax.experimental import pallas as pl
from jax.experimental.pallas import tpu as pltpu

N_DEV = 4


def kernel(x, w_mat, scale_x, scale_w):
    m, k_shard = x.shape
    _, n = w_mat.shape
    m_chunk = m // N_DEV
    n_half = n // 2
    n_hops = N_DEV - 1

    def body(x_ref, w_ref, sx_ref, sw_ref, out_ref,
             wb_ref, accR_ref, accL_ref, commR_ref, commL_ref,
             send_sems, recv_sems, copy_sems):
        my = lax.axis_index("i")
        left = (my + N_DEV - 1) % N_DEV
        right = (my + 1) % N_DEV

        barrier_sem = pltpu.get_barrier_semaphore()
        for nbr in (left, right):
            pl.semaphore_signal(
                barrier_sem, inc=1,
                device_id=(nbr,), device_id_type=pl.DeviceIdType.MESH,
            )
        pl.semaphore_wait(barrier_sem, 2)

        wb_ref[...] = w_ref[...].astype(jnp.bfloat16)

        def gemm(c, buf, acc, col0):
            xb = x_ref[pl.ds(c * m_chunk, m_chunk), :].astype(jnp.bfloat16)
            acc[buf] = lax.dot_general(
                xb, wb_ref[:, col0:col0 + n_half],
                (((1,), (0,)), ((), ())),
                preferred_element_type=jnp.float32,
            )

        def rdma(src, dst, dirn, h, dev):
            return pltpu.make_async_remote_copy(
                src_ref=src, dst_ref=dst,
                send_sem=send_sems.at[dirn, h],
                recv_sem=recv_sems.at[dirn, h],
                device_id=(dev,), device_id_type=pl.DeviceIdType.MESH,
            )

        gemm(my, 0, accR_ref, 0)
        gemm(my, 0, accL_ref, n_half)
        for s in range(n_hops):
            sbuf, wbuf = s % 2, (s + 1) % 2
            rR = rdma(accR_ref.at[sbuf], commR_ref.at[s], 0, s, right)
            rL = rdma(accL_ref.at[sbuf], commL_ref.at[s], 1, s, left)
            rR.start()
            rL.start()
            gemm((my - s - 1) % N_DEV, wbuf, accR_ref, 0)
            gemm((my + s + 1) % N_DEV, wbuf, accL_ref, n_half)
            rR.wait()
            accR_ref[wbuf] = accR_ref[wbuf] + commR_ref[s]
            rL.wait()
            accL_ref[wbuf] = accL_ref[wbuf] + commL_ref[s]

        fb = n_hops % 2
        own_R = (my + 1) % N_DEV
        own_L = (my + N_DEV - 1) % N_DEV

        scale = sx_ref[0] * sw_ref[0]
        yR = accR_ref[fb] * scale
        accR_ref[fb] = yR / (1.0 + jnp.exp(-jnp.clip(yR, -60.0, 60.0)))
        yL = accL_ref[fb] * scale
        accL_ref[fb] = yL / (1.0 + jnp.exp(-jnp.clip(yL, -60.0, 60.0)))

        cpR = pltpu.make_async_copy(
            accR_ref.at[fb],
            out_ref.at[pl.ds(own_R * m_chunk, m_chunk), pl.ds(0, n_half)],
            copy_sems.at[0],
        )
        cpL = pltpu.make_async_copy(
            accL_ref.at[fb],
            out_ref.at[pl.ds(own_L * m_chunk, m_chunk), pl.ds(n_half, n_half)],
            copy_sems.at[1],
        )
        cpR.start()
        cpL.start()

        for t in range(n_hops):
            h = n_hops + t
            cR = (my + 1 - t) % N_DEV
            cL = (my + N_DEV - 1 + t) % N_DEV
            dstR = out_ref.at[pl.ds(cR * m_chunk, m_chunk), pl.ds(0, n_half)]
            dstL = out_ref.at[pl.ds(cL * m_chunk, m_chunk),
                              pl.ds(n_half, n_half)]
            srcR = accR_ref.at[fb] if t == 0 else dstR
            srcL = accL_ref.at[fb] if t == 0 else dstL
            rR = rdma(srcR, dstR, 0, h, right)
            rL = rdma(srcL, dstL, 1, h, left)
            rR.start()
            rL.start()
            rR.wait()
            rL.wait()

        cpR.wait()
        cpL.wait()

    return pl.pallas_call(
        body,
        out_shape=jax.ShapeDtypeStruct((m, n), jnp.float32),
        in_specs=[
            pl.BlockSpec(memory_space=pltpu.VMEM),
            pl.BlockSpec(memory_space=pltpu.VMEM),
            pl.BlockSpec(memory_space=pltpu.SMEM),
            pl.BlockSpec(memory_space=pltpu.SMEM),
        ],
        out_specs=pl.BlockSpec(memory_space=pltpu.MemorySpace.HBM),
        scratch_shapes=[
            pltpu.VMEM((k_shard, n), jnp.bfloat16),
            pltpu.VMEM((2, m_chunk, n_half), jnp.float32),
            pltpu.VMEM((2, m_chunk, n_half), jnp.float32),
            pltpu.VMEM((n_hops, m_chunk, n_half), jnp.float32),
            pltpu.VMEM((n_hops, m_chunk, n_half), jnp.float32),
            pltpu.SemaphoreType.DMA((2, 2 * n_hops)),
            pltpu.SemaphoreType.DMA((2, 2 * n_hops)),
            pltpu.SemaphoreType.DMA((2,)),
        ],
        compiler_params=pltpu.CompilerParams(
            collective_id=0,
            vmem_limit_bytes=56 * 1024 * 1024,
        ),
    )(x, w_mat, scale_x, scale_w)


# device time: 305681 ns/iter; 1.9369x vs baseline; 1.0445x over previous
import jax
import jax.numpy as jnp
from jax import lax
from jax.experimental import pallas as pl
from jax.experimental.pallas import tpu as pltpu

N_DEV = 4


def kernel(x, w_mat, scale_x, scale_w):
    m, k_shard = x.shape
    _, n = w_mat.shape
    m_chunk = m // N_DEV
    n_half = n // 2
    sub = n_half // 2
    n_hops = N_DEV - 1

    def body(x_ref, w_ref, sx_ref, sw_ref, out_ref,
             wb_ref, accR_ref, accL_ref, commR_ref, commL_ref,
             send_sems, recv_sems, copy_sems):
        my = lax.axis_index("i")
        left = (my + N_DEV - 1) % N_DEV
        right = (my + 1) % N_DEV

        barrier_sem = pltpu.get_barrier_semaphore()
        for nbr in (left, right):
            pl.semaphore_signal(
                barrier_sem, inc=1,
                device_id=(nbr,), device_id_type=pl.DeviceIdType.MESH,
            )
        pl.semaphore_wait(barrier_sem, 2)

        wb_ref[...] = w_ref[...].astype(jnp.bfloat16)

        acc = (accR_ref, accL_ref)
        comm = (commR_ref, commL_ref)
        col0 = (0, n_half)
        dev = (right, left)

        def gemm(c, buf, d):
            xb = x_ref[pl.ds(c * m_chunk, m_chunk), :].astype(jnp.bfloat16)
            acc[d][buf] = lax.dot_general(
                xb, wb_ref[:, col0[d]:col0[d] + n_half],
                (((1,), (0,)), ((), ())),
                preferred_element_type=jnp.float32,
            )

        def rs_rdma(s, h, d):
            cs = slice(h * sub, (h + 1) * sub)
            return pltpu.make_async_remote_copy(
                src_ref=acc[d].at[s % 2, :, cs],
                dst_ref=comm[d].at[s, :, cs],
                send_sem=send_sems.at[d, s, h],
                recv_sem=recv_sems.at[d, s, h],
                device_id=(dev[d],), device_id_type=pl.DeviceIdType.MESH,
            )

        def ag_rdma(t, h, d):
            cR = (my + 1 - t) % N_DEV if d == 0 else (my + N_DEV - 1 + t) % N_DEV
            dst = out_ref.at[pl.ds(cR * m_chunk, m_chunk),
                             pl.ds(col0[d] + h * sub, sub)]
            src = acc[d].at[n_hops % 2, :, h * sub:(h + 1) * sub] if t == 0 else dst
            return pltpu.make_async_remote_copy(
                src_ref=src, dst_ref=dst,
                send_sem=send_sems.at[d, n_hops + t, h],
                recv_sem=recv_sems.at[d, n_hops + t, h],
                device_id=(dev[d],), device_id_type=pl.DeviceIdType.MESH,
            )

        scale = sx_ref[0] * sw_ref[0]
        fb = n_hops % 2

        gemm(my, 0, 0)
        gemm(my, 0, 1)
        for d in (0, 1):
            for h in (0, 1):
                rs_rdma(0, h, d).start()

        for s in range(n_hops):
            wbuf = (s + 1) % 2
            if s >= 1:
                for d in (0, 1):
                    for h in (0, 1):
                        rs_rdma(s - 1, h, d).wait_send()
            gemm((my - s - 1) % N_DEV, wbuf, 0)
            gemm((my + s + 1) % N_DEV, wbuf, 1)
            for h in (0, 1):
                for d in (0, 1):
                    cs = slice(h * sub, (h + 1) * sub)
                    rs_rdma(s, h, d).wait_recv()
                    acc[d][wbuf, :, cs] = (
                        acc[d][wbuf, :, cs] + comm[d][s, :, cs]
                    )
                    if s < n_hops - 1:
                        rs_rdma(s + 1, h, d).start()
                    else:
                        y = acc[d][fb, :, cs] * scale
                        acc[d][fb, :, cs] = y / (
                            1.0 + jnp.exp(-jnp.clip(y, -60.0, 60.0))
                        )
                        ag_rdma(0, h, d).start()

        own_c = ((my + 1) % N_DEV, (my + N_DEV - 1) % N_DEV)
        cps = []
        for d in (0, 1):
            cp = pltpu.make_async_copy(
                acc[d].at[fb],
                out_ref.at[pl.ds(own_c[d] * m_chunk, m_chunk),
                           pl.ds(col0[d], n_half)],
                copy_sems.at[d],
            )
            cp.start()
            cps.append(cp)

        for t in range(n_hops):
            for h in (0, 1):
                for d in (0, 1):
                    ag_rdma(t, h, d).wait_recv()
                    if t < n_hops - 1:
                        ag_rdma(t + 1, h, d).start()

        for d in (0, 1):
            for h in (0, 1):
                rs_rdma(n_hops - 1, h, d).wait_send()
                for t in range(n_hops):
                    ag_rdma(t, h, d).wait_send()
        for cp in cps:
            cp.wait()

    return pl.pallas_call(
        body,
        out_shape=jax.ShapeDtypeStruct((m, n), jnp.float32),
        in_specs=[
            pl.BlockSpec(memory_space=pltpu.VMEM),
            pl.BlockSpec(memory_space=pltpu.VMEM),
            pl.BlockSpec(memory_space=pltpu.SMEM),
            pl.BlockSpec(memory_space=pltpu.SMEM),
        ],
        out_specs=pl.BlockSpec(memory_space=pltpu.MemorySpace.HBM),
        scratch_shapes=[
            pltpu.VMEM((k_shard, n), jnp.bfloat16),
            pltpu.VMEM((2, m_chunk, n_half), jnp.float32),
            pltpu.VMEM((2, m_chunk, n_half), jnp.float32),
            pltpu.VMEM((n_hops, m_chunk, n_half), jnp.float32),
            pltpu.VMEM((n_hops, m_chunk, n_half), jnp.float32),
            pltpu.SemaphoreType.DMA((2, 2 * n_hops, 2)),
            pltpu.SemaphoreType.DMA((2, 2 * n_hops, 2)),
            pltpu.SemaphoreType.DMA((2,)),
        ],
        compiler_params=pltpu.CompilerParams(
            collective_id=0,
            vmem_limit_bytes=56 * 1024 * 1024,
        ),
    )(x, w_mat, scale_x, scale_w)
